# baseline (speedup 1.0000x reference)
import jax
import jax.numpy as jnp
from jax import lax
from jax.experimental import pallas as pl
from jax.experimental.pallas import tpu as pltpu
from jax.experimental.pallas import tpu_sc as plsc

_B = 16384
_D = 128
_K = _D // 16
_NC = 2
_NS = 16
_NW = _NC * _NS
_RPW = _B // _NW
_C = 64
_NCH = _RPW // _C


def _rsqrt_nr(x):
    xi = lax.bitcast_convert_type(x, jnp.int32)
    yi = jnp.int32(0x5F3759DF) - lax.shift_right_logical(xi, 1)
    y = lax.bitcast_convert_type(yi, jnp.float32)
    hx = x * jnp.float32(0.5)
    for _ in range(3):
        y = y * (jnp.float32(1.5) - hx * y * y)
    return y


def _splat(s):
    return lax.broadcast_in_dim(s, (16,), ())


def _body(i2, i3, i4, i5, i6, t2, d3, d4, d5, d6, wb,
          out_hbm,
          ix0, ix1, ix2, ix3, ix4, g0, g1, g2, g3, g4, out_v, wb_v,
          s0, s1, s2, s3, s4):
    wid = lax.axis_index("s") * _NC + lax.axis_index("c")
    base = wid * _RPW

    idx_refs = (ix0, ix1, ix2, ix3, ix4)
    for idx_hbm, idx_v in zip((i2, i3, i4, i5, i6), idx_refs):
        pltpu.sync_copy(idx_hbm.at[pl.ds(base, _RPW)], idx_v)
    pltpu.sync_copy(wb, wb_v)

    tables = (t2, d3, d4, d5, d6)
    gbufs = (g0, g1, g2, g3, g4)
    sems = (s0, s1, s2, s3, s4)

    w_regs = [wb_v[pl.ds(k * 16, 16)] for k in range(_K)]
    b_splat = _splat(wb_v[pl.ds(_D, 16)][0])
    iota16 = lax.broadcasted_iota(jnp.int32, (16,), 0)

    def chunk_body(ch, _):
        cb = pl.multiple_of(ch * _C, _C)
        copies = []
        for l in range(5):
            copies.append(pltpu.async_copy(
                tables[l].at[idx_refs[l].at[pl.ds(cb, _C)]], gbufs[l], sems[l]))
        for c in copies:
            c.wait()

        def group_body(g, _):
            out_vec = jnp.zeros((16,), jnp.float32)
            for j in range(16):
                r = g * 16 + j
                u = [g0[r, pl.ds(k * 16, 16)] for k in range(_K)]
                acc = u[0] * u[0]
                for k in range(1, _K):
                    acc = acc + u[k] * u[k]
                y = _rsqrt_nr(_splat(jnp.sum(acc)))
                for l in range(1, 5):
                    gb = gbufs[l]
                    u = [y * u[k] + gb[r, pl.ds(k * 16, 16)] for k in range(_K)]
                    acc = u[0] * u[0]
                    for k in range(1, _K):
                        acc = acc + u[k] * u[k]
                    y = _rsqrt_nr(_splat(jnp.sum(acc)))
                dotv = u[0] * w_regs[0]
                for k in range(1, _K):
                    dotv = dotv + u[k] * w_regs[k]
                row_out = y * _splat(jnp.sum(dotv)) + b_splat
                out_vec = jnp.where(iota16 == j, row_out, out_vec)
            out_v[pl.ds(pl.multiple_of(ch * _C + g * 16, 16), 16)] = out_vec
            return 0

        lax.fori_loop(0, _C // 16, group_body, 0)
        return 0

    lax.fori_loop(0, _NCH, chunk_body, 0)
    pltpu.sync_copy(out_v, out_hbm.at[pl.ds(base, _RPW)])


def kernel(naics_2_digit, naics_3_digit, naics_4_digit, naics_5_digit, naics_6_digit,
           table2, delta3, delta4, delta5, delta6, W, b):
    wb = jnp.concatenate([W.reshape(_D), b, jnp.zeros((15,), jnp.float32)])
    mesh = plsc.VectorSubcoreMesh(core_axis_name="c", subcore_axis_name="s")
    scratch = [pltpu.VMEM((_RPW,), jnp.int32)] * 5 + [
        pltpu.VMEM((_C, _D), jnp.float32)] * 5 + [
        pltpu.VMEM((_RPW,), jnp.float32),
        pltpu.VMEM((_D + 16,), jnp.float32),
    ] + [pltpu.SemaphoreType.DMA] * 5
    call = pl.kernel(
        _body,
        out_type=jax.ShapeDtypeStruct((_B,), jnp.float32),
        mesh=mesh,
        scratch_types=scratch,
        compiler_params=pltpu.CompilerParams(needs_layout_passes=False),
    )
    out = call(naics_2_digit, naics_3_digit, naics_4_digit, naics_5_digit,
               naics_6_digit, table2, delta3, delta4, delta5, delta6, wb)
    return out.reshape(_B, 1)
